# Initial kernel scaffold; baseline (speedup 1.0000x reference)
import jax
import jax.numpy as jnp
from jax.experimental import pallas as pl


def kernel(x, edge_index, W, b_lin, att, bias):
    raise NotImplementedError("write your pallas kernel here")


# speedup vs baseline: 9.9686x; 9.9686x over previous
import functools

import jax
import jax.numpy as jnp
from jax import lax
from jax.experimental import pallas as pl
from jax.experimental.pallas import tpu as pltpu
from jax.experimental.pallas import tpu_sc as plsc

N = 10000
E = 320000
H = 4
C = 128
HC = H * C
ACC_W = HC + 16
EP = E + N
NSUB = 16
EPT = 20640
EPAD = NSUB * EPT
SB = 2064
SCN = SB // 16
RANGES = ((0, 2496, 0), (2496, 2496, 0), (4992, 2496, 1), (7488, 2512, 1))
RMAX = 2512
MMB = 1000


def _mm_body(x_ref, w_ref, b_ref, o_ref):
    o_ref[...] = (
        jnp.dot(x_ref[...], w_ref[...], preferred_element_type=jnp.float32)
        + b_ref[...]
    )


def _linear(x, W, b):
    return pl.pallas_call(
        _mm_body,
        grid=(N // MMB,),
        in_specs=[
            pl.BlockSpec((MMB, C), lambda i: (i, 0)),
            pl.BlockSpec((C, HC), lambda i: (0, 0)),
            pl.BlockSpec((1, HC), lambda i: (0, 0)),
        ],
        out_specs=pl.BlockSpec((MMB, HC), lambda i: (i, 0)),
        out_shape=jax.ShapeDtypeStruct((N, HC), jnp.float32),
    )(x, W, b.reshape(1, HC))


def _sc_body(xl, srcp, dstp, att, zeros, acc_out,
             att_v, sblk_s, sblk_d, csrc, cdst,
             xi_v, xj_v, sbuf, idx_s, idx_d, idx_w,
             acc_sh, sem_a, sem_b):
    core = lax.axis_index("c")
    sub = lax.axis_index("s")
    pltpu.sync_copy(att, att_v)
    iota16 = lax.iota(jnp.int32, 16)

    for base, size, owner in RANGES:
        CH = 160
        rem = size - 15 * CH

        @pl.when(core == owner)
        def _range():
            @pl.when(sub < 15)
            def _():
                pltpu.sync_copy(zeros.at[pl.ds(sub * CH, CH)],
                                acc_sh.at[pl.ds(sub * CH, CH)])

            @pl.when(sub == 15)
            def _():
                pltpu.sync_copy(zeros.at[pl.ds(15 * CH, rem)],
                                acc_sh.at[pl.ds(15 * CH, rem)])

            plsc.subcore_barrier()

            def proc(j, ncomp):
                lanes = j * 16 + iota16
                ml = lanes < ncomp
                sv = jnp.where(ml, csrc[pl.ds(j * 16, 16)], 0)
                dv = jnp.where(ml, cdst[pl.ds(j * 16, 16)], 0)
                idx_s[...] = sv
                idx_d[...] = jnp.where(ml, dv, base)
                idx_w[...] = jnp.where(ml, dv - base, 0)
                cp1 = pltpu.async_copy(xl.at[idx_s], xj_v, sem_a)
                cp2 = pltpu.async_copy(xl.at[idx_d], xi_v, sem_b)
                cp1.wait()
                cp2.wait()

                def edge(e, carry):
                    live = (j * 16 + e) < ncomp
                    pm = jnp.where(live, 1.0, 0.0)
                    pdv = jnp.zeros((16,), jnp.float32)
                    for h in range(H):
                        acc = jnp.zeros((16,), jnp.float32)
                        for k in range(C // 16):
                            off = h * C + k * 16
                            s = xi_v[e, pl.ds(off, 16)] + xj_v[e, pl.ds(off, 16)]
                            l = jnp.maximum(s, 0.0) + 0.2 * jnp.minimum(s, 0.0)
                            acc = acc + l * att_v[h, pl.ds(k * 16, 16)]
                        a_h = jnp.sum(acc)
                        pv = jnp.exp(jnp.full((16,), a_h)) * pm
                        for k in range(C // 16):
                            off = h * C + k * 16
                            sbuf[e, pl.ds(off, 16)] = xj_v[e, pl.ds(off, 16)] * pv
                        pdv = jnp.where(iota16 == h, pv, pdv)
                    sbuf[e, pl.ds(HC, 16)] = pdv
                    return carry

                lax.fori_loop(0, 16, edge, 0)
                pltpu.sync_copy(sbuf, acc_sh.at[idx_w], add=True)
                return ncomp

            def scan_block(b, carry):
                off = sub * EPT + b * SB
                pltpu.sync_copy(srcp.at[pl.ds(off, SB)], sblk_s)
                pltpu.sync_copy(dstp.at[pl.ds(off, SB)], sblk_d)

                def scan_chunk(i, cur):
                    s16 = sblk_s[pl.ds(i * 16, 16)]
                    d16 = sblk_d[pl.ds(i * 16, 16)]
                    m = (d16 >= base) & (d16 < base + size)
                    mi = jnp.where(m, 1, 0)
                    pos = cur + jnp.cumsum(mi) - 1
                    plsc.store_scatter(csrc, [pos], s16, mask=m)
                    plsc.store_scatter(cdst, [pos], d16, mask=m)
                    return cur + jnp.sum(mi)

                ncomp = lax.fori_loop(0, SCN, scan_chunk, jnp.int32(0))
                nch = (ncomp + 15) // 16
                lax.fori_loop(0, nch, proc, ncomp)
                return carry

            lax.fori_loop(0, EPT // SB, scan_block, 0)
            plsc.subcore_barrier()

            @pl.when(sub < 15)
            def _():
                pltpu.sync_copy(acc_sh.at[pl.ds(sub * CH, CH)],
                                acc_out.at[pl.ds(base + sub * CH, CH)])

            @pl.when(sub == 15)
            def _():
                pltpu.sync_copy(acc_sh.at[pl.ds(15 * CH, rem)],
                                acc_out.at[pl.ds(base + 15 * CH, rem)])

            plsc.subcore_barrier()


_sc_edge = functools.partial(
    pl.kernel,
    mesh=plsc.VectorSubcoreMesh(core_axis_name="c", subcore_axis_name="s"),
    compiler_params=pltpu.CompilerParams(
        needs_layout_passes=False, use_tc_tiling_on_sc=False),
    out_type=jax.ShapeDtypeStruct((N, ACC_W), jnp.float32),
    scratch_types=[
        pltpu.VMEM((H, C), jnp.float32),
        pltpu.VMEM((SB,), jnp.int32),
        pltpu.VMEM((SB,), jnp.int32),
        pltpu.VMEM((SB + 16,), jnp.int32),
        pltpu.VMEM((SB + 16,), jnp.int32),
        pltpu.VMEM((16, HC), jnp.float32),
        pltpu.VMEM((16, HC), jnp.float32),
        pltpu.VMEM((16, ACC_W), jnp.float32),
        pltpu.VMEM((16,), jnp.int32),
        pltpu.VMEM((16,), jnp.int32),
        pltpu.VMEM((16,), jnp.int32),
        pltpu.VMEM_SHARED((RMAX, ACC_W), jnp.float32),
        pltpu.SemaphoreType.DMA,
        pltpu.SemaphoreType.DMA,
    ],
)(_sc_body)


def _ep_body(a_ref, b_ref, o_ref):
    a = a_ref[...]
    for h in range(H):
        num = a[:, h * C:(h + 1) * C]
        den = a[:, HC + h:HC + h + 1]
        o_ref[:, h * C:(h + 1) * C] = jnp.maximum(
            num / jnp.maximum(den, 1e-30) + b_ref[:, h * C:(h + 1) * C], 0.0)


def _epilogue(acc, bias):
    return pl.pallas_call(
        _ep_body,
        grid=(N // MMB,),
        in_specs=[
            pl.BlockSpec((MMB, ACC_W), lambda i: (i, 0)),
            pl.BlockSpec((1, HC), lambda i: (0, 0)),
        ],
        out_specs=pl.BlockSpec((MMB, HC), lambda i: (i, 0)),
        out_shape=jax.ShapeDtypeStruct((N, HC), jnp.float32),
    )(acc, bias.reshape(1, HC))


def kernel(x, edge_index, W, b_lin, att, bias):
    xl = _linear(x, W, b_lin)
    loop = jnp.arange(N, dtype=jnp.int32)
    pad = EPAD - EP
    srcp = jnp.concatenate([edge_index[0], loop,
                            jnp.zeros((pad,), jnp.int32)])
    dstp = jnp.concatenate([edge_index[1], loop,
                            jnp.full((pad,), N, jnp.int32)])
    zeros = jnp.zeros((RMAX, ACC_W), jnp.float32)
    acc = _sc_edge(xl, srcp, dstp, att, zeros)
    return _epilogue(acc, bias)


# speedup vs baseline: 14.6507x; 1.4697x over previous
import functools

import jax
import jax.numpy as jnp
from jax import lax
from jax.experimental import pallas as pl
from jax.experimental.pallas import tpu as pltpu
from jax.experimental.pallas import tpu_sc as plsc

N = 10000
E = 320000
H = 4
C = 128
HC = H * C
ACC_W = HC + 16
EP = E + N
NSUB = 16
EPT = 20640
EPAD = NSUB * EPT
SB = 2064
SCN = SB // 16
RANGES = ((0, 1664, 0), (1664, 1664, 0), (3328, 1664, 0),
          (4992, 1664, 1), (6656, 1664, 1), (8320, 1680, 1))
RMAX = 1680
MMB = 1000


def _mm_body(x_ref, w_ref, b_ref, o_ref):
    o_ref[...] = (
        jnp.dot(x_ref[...], w_ref[...], preferred_element_type=jnp.float32)
        + b_ref[...]
    )


def _linear(x, W, b):
    return pl.pallas_call(
        _mm_body,
        grid=(N // MMB,),
        in_specs=[
            pl.BlockSpec((MMB, C), lambda i: (i, 0)),
            pl.BlockSpec((C, HC), lambda i: (0, 0)),
            pl.BlockSpec((1, HC), lambda i: (0, 0)),
        ],
        out_specs=pl.BlockSpec((MMB, HC), lambda i: (i, 0)),
        out_shape=jax.ShapeDtypeStruct((N, HC), jnp.float32),
    )(x, W, b.reshape(1, HC))


def _sc_body(xl, srcp, dstp, att, zeros, acc_out,
             att_v, sblk_s, sblk_d, csrc, cdst,
             xi0, xi1, xj0, xj1, sbuf,
             ids0, ids1, idd0, idd1, idw0, idw1,
             acc_sh, sa0, sa1, sb0, sb1):
    core = lax.axis_index("c")
    sub = lax.axis_index("s")
    pltpu.sync_copy(att, att_v)
    iota16 = lax.iota(jnp.int32, 16)
    xi = (xi0, xi1)
    xj = (xj0, xj1)
    ids = (ids0, ids1)
    idd = (idd0, idd1)
    idw = (idw0, idw1)
    sa = (sa0, sa1)
    sb = (sb0, sb1)

    for base, size, owner in RANGES:
        rows = size // NSUB

        @pl.when(core == owner)
        def _range():
            pltpu.sync_copy(zeros.at[pl.ds(sub * rows, rows)],
                            acc_sh.at[pl.ds(sub * rows, rows)])
            plsc.subcore_barrier()

            def prep_fire(jv, b, ncomp):
                lanes = jv * 16 + iota16
                ml = lanes < ncomp
                sv = jnp.where(ml, csrc[pl.ds(jv * 16, 16)], 0)
                dv = jnp.where(ml, cdst[pl.ds(jv * 16, 16)], base)
                ids[b][...] = sv
                idd[b][...] = dv
                idw[b][...] = dv - base
                pltpu.async_copy(xl.at[ids[b]], xj[b], sa[b])
                pltpu.async_copy(xl.at[idd[b]], xi[b], sb[b])

            def compute_chunk(jv, b, ncomp):
                pltpu.make_async_copy(xl.at[ids[b]], xj[b], sa[b]).wait()
                pltpu.make_async_copy(xl.at[idd[b]], xi[b], sb[b]).wait()

                def edge(e, carry):
                    live = (jv * 16 + e) < ncomp
                    pm = jnp.where(live, 1.0, 0.0)
                    pdv = jnp.zeros((16,), jnp.float32)
                    for h in range(H):
                        acc = jnp.zeros((16,), jnp.float32)
                        xjc = []
                        for k in range(C // 16):
                            off = h * C + k * 16
                            xjk = xj[b][e, pl.ds(off, 16)]
                            xjc.append(xjk)
                            s = xi[b][e, pl.ds(off, 16)] + xjk
                            l = (jnp.maximum(s, 0.0)
                                 + 0.2 * jnp.minimum(s, 0.0))
                            acc = acc + l * att_v[h, pl.ds(k * 16, 16)]
                        a_h = jnp.sum(acc)
                        pv = jnp.exp(jnp.full((16,), a_h)) * pm
                        for k in range(C // 16):
                            off = h * C + k * 16
                            sbuf[e, pl.ds(off, 16)] = xjc[k] * pv
                        pdv = jnp.where(iota16 == h, pv, pdv)
                    sbuf[e, pl.ds(HC, 16)] = pdv
                    return carry

                lax.fori_loop(0, 16, edge, 0)
                pltpu.sync_copy(sbuf, acc_sh.at[idw[b]], add=True)

            def scan_block(blk, carry):
                off = sub * EPT + blk * SB
                pltpu.sync_copy(srcp.at[pl.ds(off, SB)], sblk_s)
                pltpu.sync_copy(dstp.at[pl.ds(off, SB)], sblk_d)

                def scan_chunk(i, cur):
                    s16 = sblk_s[pl.ds(i * 16, 16)]
                    d16 = sblk_d[pl.ds(i * 16, 16)]
                    m = (d16 >= base) & (d16 < base + size)
                    mi = jnp.where(m, 1, 0)
                    pos = cur + jnp.cumsum(mi) - 1
                    plsc.store_scatter(csrc, [pos], s16, mask=m)
                    plsc.store_scatter(cdst, [pos], d16, mask=m)
                    return cur + jnp.sum(mi)

                ncomp = lax.fori_loop(0, SCN, scan_chunk, jnp.int32(0))

                @pl.when(ncomp > 0)
                def _():
                    prep_fire(jnp.int32(0), 0, ncomp)

                    def outer(jj, carry2):
                        for b in range(2):
                            jv = jj * 2 + b
                            prep_fire(jv + 1, 1 - b, ncomp)
                            compute_chunk(jv, b, ncomp)
                        return carry2

                    nch2 = (ncomp + 31) // 32
                    lax.fori_loop(0, nch2, outer, 0)
                    b_last = 0
                    pltpu.make_async_copy(
                        xl.at[ids[b_last]], xj[b_last], sa[b_last]).wait()
                    pltpu.make_async_copy(
                        xl.at[idd[b_last]], xi[b_last], sb[b_last]).wait()

                return carry

            lax.fori_loop(0, EPT // SB, scan_block, 0)
            plsc.subcore_barrier()
            pltpu.sync_copy(acc_sh.at[pl.ds(sub * rows, rows)],
                            acc_out.at[pl.ds(base + sub * rows, rows)])
            plsc.subcore_barrier()


_sc_edge = functools.partial(
    pl.kernel,
    mesh=plsc.VectorSubcoreMesh(core_axis_name="c", subcore_axis_name="s"),
    compiler_params=pltpu.CompilerParams(
        needs_layout_passes=False, use_tc_tiling_on_sc=False),
    out_type=jax.ShapeDtypeStruct((N, ACC_W), jnp.float32),
    scratch_types=[
        pltpu.VMEM((H, C), jnp.float32),
        pltpu.VMEM((SB,), jnp.int32),
        pltpu.VMEM((SB,), jnp.int32),
        pltpu.VMEM((SB + 32,), jnp.int32),
        pltpu.VMEM((SB + 32,), jnp.int32),
        pltpu.VMEM((16, HC), jnp.float32),
        pltpu.VMEM((16, HC), jnp.float32),
        pltpu.VMEM((16, HC), jnp.float32),
        pltpu.VMEM((16, HC), jnp.float32),
        pltpu.VMEM((16, ACC_W), jnp.float32),
        pltpu.VMEM((16,), jnp.int32),
        pltpu.VMEM((16,), jnp.int32),
        pltpu.VMEM((16,), jnp.int32),
        pltpu.VMEM((16,), jnp.int32),
        pltpu.VMEM((16,), jnp.int32),
        pltpu.VMEM((16,), jnp.int32),
        pltpu.VMEM_SHARED((RMAX, ACC_W), jnp.float32),
        pltpu.SemaphoreType.DMA,
        pltpu.SemaphoreType.DMA,
        pltpu.SemaphoreType.DMA,
        pltpu.SemaphoreType.DMA,
    ],
)(_sc_body)


def _ep_body(a_ref, b_ref, o_ref):
    a = a_ref[...]
    for h in range(H):
        num = a[:, h * C:(h + 1) * C]
        den = a[:, HC + h:HC + h + 1]
        o_ref[:, h * C:(h + 1) * C] = jnp.maximum(
            num / jnp.maximum(den, 1e-30) + b_ref[:, h * C:(h + 1) * C], 0.0)


def _epilogue(acc, bias):
    return pl.pallas_call(
        _ep_body,
        grid=(N // MMB,),
        in_specs=[
            pl.BlockSpec((MMB, ACC_W), lambda i: (i, 0)),
            pl.BlockSpec((1, HC), lambda i: (0, 0)),
        ],
        out_specs=pl.BlockSpec((MMB, HC), lambda i: (i, 0)),
        out_shape=jax.ShapeDtypeStruct((N, HC), jnp.float32),
    )(acc, bias.reshape(1, HC))


def kernel(x, edge_index, W, b_lin, att, bias):
    xl = _linear(x, W, b_lin)
    loop = jnp.arange(N, dtype=jnp.int32)
    pad = EPAD - EP
    srcp = jnp.concatenate([edge_index[0], loop,
                            jnp.zeros((pad,), jnp.int32)])
    dstp = jnp.concatenate([edge_index[1], loop,
                            jnp.full((pad,), N, jnp.int32)])
    zeros = jnp.zeros((RMAX, ACC_W), jnp.float32)
    acc = _sc_edge(xl, srcp, dstp, att, zeros)
    return _epilogue(acc, bias)


# speedup vs baseline: 15.6088x; 1.0654x over previous
import functools

import jax
import jax.numpy as jnp
from jax import lax
from jax.experimental import pallas as pl
from jax.experimental.pallas import tpu as pltpu
from jax.experimental.pallas import tpu_sc as plsc

N = 10000
E = 320000
H = 4
C = 128
HC = H * C
ACC_W = HC + 16
EP = E + N
NSUB = 16
EPT = 20640
EPAD = NSUB * EPT
SB = 2064
SCN = SB // 16
RANGES = ((0, 1664, 0), (1664, 1664, 0), (3328, 1664, 0),
          (4992, 1664, 1), (6656, 1664, 1), (8320, 1680, 1))
RMAX = 1680
MMB = 1000


def _mm_body(x_ref, w_ref, b_ref, o_ref):
    o_ref[...] = (
        jnp.dot(x_ref[...], w_ref[...], preferred_element_type=jnp.float32)
        + b_ref[...]
    )


def _linear(x, W, b):
    return pl.pallas_call(
        _mm_body,
        grid=(N // MMB,),
        in_specs=[
            pl.BlockSpec((MMB, C), lambda i: (i, 0)),
            pl.BlockSpec((C, HC), lambda i: (0, 0)),
            pl.BlockSpec((1, HC), lambda i: (0, 0)),
        ],
        out_specs=pl.BlockSpec((MMB, HC), lambda i: (i, 0)),
        out_shape=jax.ShapeDtypeStruct((N, HC), jnp.float32),
    )(x, W, b.reshape(1, HC))


def _sc_body(xl, srcp, dstp, att, zeros, acc_out,
             att_v, sblk_s, sblk_d, csrc, cdst,
             gb0, gb1, sb0, sb1,
             idc0, idc1, idw0, idw1,
             acc_sh, sa0, sa1, ss0, ss1):
    core = lax.axis_index("c")
    sub = lax.axis_index("s")
    pltpu.sync_copy(att, att_v)
    iota16 = lax.iota(jnp.int32, 16)
    gb = (gb0, gb1)
    sbuf = (sb0, sb1)
    idc = (idc0, idc1)
    idw = (idw0, idw1)
    sa = (sa0, sa1)
    ss = (ss0, ss1)
    attr = [[att_v[h, pl.ds(k * 16, 16)] for k in range(C // 16)]
            for h in range(H)]

    for base, size, owner in RANGES:
        rows = size // NSUB

        @pl.when(core == owner)
        def _range():
            pltpu.sync_copy(zeros.at[pl.ds(sub * rows, rows)],
                            acc_sh.at[pl.ds(sub * rows, rows)])
            plsc.subcore_barrier()

            def prep_fire(jv, b, ncomp):
                lanes = jv * 16 + iota16
                ml = lanes < ncomp
                sv = jnp.where(ml, csrc[pl.ds(jv * 16, 16)], 0)
                dv = jnp.where(ml, cdst[pl.ds(jv * 16, 16)], base)
                idc[b][pl.ds(0, 16)] = sv
                idc[b][pl.ds(16, 16)] = dv
                pltpu.async_copy(xl.at[idc[b]], gb[b], sa[b])

            def compute_chunk(jv, b, ncomp):
                pltpu.make_async_copy(xl.at[idc[b]], gb[b], sa[b]).wait()

                @pl.when(jv >= 2)
                def _():
                    pltpu.make_async_copy(
                        sbuf[b], acc_sh.at[idw[b]], ss[b]).wait()

                lanes = jv * 16 + iota16
                ml = lanes < ncomp
                dv = jnp.where(ml, cdst[pl.ds(jv * 16, 16)], base)
                idw[b][...] = dv - base

                def edge(e, carry):
                    live = (jv * 16 + e) < ncomp
                    pm = jnp.where(live, 1.0, 0.0)
                    pdv = jnp.zeros((16,), jnp.float32)
                    for h in range(H):
                        acc = jnp.zeros((16,), jnp.float32)
                        xjc = []
                        for k in range(C // 16):
                            off = h * C + k * 16
                            xjk = gb[b][e, pl.ds(off, 16)]
                            xjc.append(xjk)
                            s = gb[b][16 + e, pl.ds(off, 16)] + xjk
                            l = (jnp.maximum(s, 0.0)
                                 + 0.2 * jnp.minimum(s, 0.0))
                            acc = acc + l * attr[h][k]
                        a_h = jnp.sum(acc)
                        pv = jnp.exp(jnp.full((16,), a_h)) * pm
                        for k in range(C // 16):
                            off = h * C + k * 16
                            sbuf[b][e, pl.ds(off, 16)] = xjc[k] * pv
                        pdv = jnp.where(iota16 == h, pv, pdv)
                    sbuf[b][e, pl.ds(HC, 16)] = pdv
                    return carry

                lax.fori_loop(0, 16, edge, 0)
                pltpu.async_copy(sbuf[b], acc_sh.at[idw[b]], ss[b], add=True)

            def scan_block(blk, carry):
                off = sub * EPT + blk * SB
                pltpu.sync_copy(srcp.at[pl.ds(off, SB)], sblk_s)
                pltpu.sync_copy(dstp.at[pl.ds(off, SB)], sblk_d)

                def scan_chunk(i, cur):
                    s16 = sblk_s[pl.ds(i * 16, 16)]
                    d16 = sblk_d[pl.ds(i * 16, 16)]
                    m = (d16 >= base) & (d16 < base + size)
                    mi = jnp.where(m, 1, 0)
                    pos = cur + jnp.cumsum(mi) - 1
                    plsc.store_scatter(csrc, [pos], s16, mask=m)
                    plsc.store_scatter(cdst, [pos], d16, mask=m)
                    return cur + jnp.sum(mi)

                ncomp = lax.fori_loop(0, SCN, scan_chunk, jnp.int32(0))

                @pl.when(ncomp > 0)
                def _():
                    prep_fire(jnp.int32(0), 0, ncomp)

                    def outer(jj, carry2):
                        for b in range(2):
                            jv = jj * 2 + b
                            prep_fire(jv + 1, 1 - b, ncomp)
                            compute_chunk(jv, b, ncomp)
                        return carry2

                    nch2 = (ncomp + 31) // 32
                    lax.fori_loop(0, nch2, outer, 0)
                    pltpu.make_async_copy(xl.at[idc[0]], gb[0], sa[0]).wait()
                    pltpu.make_async_copy(
                        sbuf[0], acc_sh.at[idw[0]], ss[0]).wait()
                    pltpu.make_async_copy(
                        sbuf[1], acc_sh.at[idw[1]], ss[1]).wait()

                return carry

            lax.fori_loop(0, EPT // SB, scan_block, 0)
            plsc.subcore_barrier()
            pltpu.sync_copy(acc_sh.at[pl.ds(sub * rows, rows)],
                            acc_out.at[pl.ds(base + sub * rows, rows)])
            plsc.subcore_barrier()


_sc_edge = functools.partial(
    pl.kernel,
    mesh=plsc.VectorSubcoreMesh(core_axis_name="c", subcore_axis_name="s"),
    compiler_params=pltpu.CompilerParams(
        needs_layout_passes=False, use_tc_tiling_on_sc=False),
    out_type=jax.ShapeDtypeStruct((N, ACC_W), jnp.float32),
    scratch_types=[
        pltpu.VMEM((H, C), jnp.float32),
        pltpu.VMEM((SB,), jnp.int32),
        pltpu.VMEM((SB,), jnp.int32),
        pltpu.VMEM((SB + 32,), jnp.int32),
        pltpu.VMEM((SB + 32,), jnp.int32),
        pltpu.VMEM((32, HC), jnp.float32),
        pltpu.VMEM((32, HC), jnp.float32),
        pltpu.VMEM((16, ACC_W), jnp.float32),
        pltpu.VMEM((16, ACC_W), jnp.float32),
        pltpu.VMEM((32,), jnp.int32),
        pltpu.VMEM((32,), jnp.int32),
        pltpu.VMEM((16,), jnp.int32),
        pltpu.VMEM((16,), jnp.int32),
        pltpu.VMEM_SHARED((RMAX, ACC_W), jnp.float32),
        pltpu.SemaphoreType.DMA,
        pltpu.SemaphoreType.DMA,
        pltpu.SemaphoreType.DMA,
        pltpu.SemaphoreType.DMA,
    ],
)(_sc_body)


def _ep_body(a_ref, b_ref, o_ref):
    a = a_ref[...]
    for h in range(H):
        num = a[:, h * C:(h + 1) * C]
        den = a[:, HC + h:HC + h + 1]
        o_ref[:, h * C:(h + 1) * C] = jnp.maximum(
            num / jnp.maximum(den, 1e-30) + b_ref[:, h * C:(h + 1) * C], 0.0)


def _epilogue(acc, bias):
    return pl.pallas_call(
        _ep_body,
        grid=(N // MMB,),
        in_specs=[
            pl.BlockSpec((MMB, ACC_W), lambda i: (i, 0)),
            pl.BlockSpec((1, HC), lambda i: (0, 0)),
        ],
        out_specs=pl.BlockSpec((MMB, HC), lambda i: (i, 0)),
        out_shape=jax.ShapeDtypeStruct((N, HC), jnp.float32),
    )(acc, bias.reshape(1, HC))


def kernel(x, edge_index, W, b_lin, att, bias):
    xl = _linear(x, W, b_lin)
    loop = jnp.arange(N, dtype=jnp.int32)
    pad = EPAD - EP
    srcp = jnp.concatenate([edge_index[0], loop,
                            jnp.zeros((pad,), jnp.int32)])
    dstp = jnp.concatenate([edge_index[1], loop,
                            jnp.full((pad,), N, jnp.int32)])
    zeros = jnp.zeros((RMAX, ACC_W), jnp.float32)
    acc = _sc_edge(xl, srcp, dstp, att, zeros)
    return _epilogue(acc, bias)
